# baseline (speedup 1.0000x reference)
import jax
import jax.numpy as jnp
from jax.experimental import pallas as pl
from jax.experimental.pallas import tpu as pltpu

M = 100000
D = 128
B = 16384
RB = 800


def _combine_body(scale_ref, var_ref, delta_ref, out_ref, pmax_ref):
    out = var_ref[...].astype(jnp.float32) * scale_ref[0] + delta_ref[...]
    out_ref[...] = out
    pmax_ref[pl.program_id(0)] = jnp.max(jnp.abs(out))


def _quant_body(scale_ref, out_ref, y_ref):
    inv = 1.0 / scale_ref[0]
    y = jnp.clip(jnp.round(out_ref[...] * inv), -128, 127)
    y_ref[...] = y.astype(jnp.int8)


def kernel(var, var_scale, indices, updates, smooth_scales):
    scaled = (updates * smooth_scales).astype(jnp.float32)
    cols = jnp.broadcast_to(jnp.arange(D), (B, D))
    delta = jnp.zeros((M, D), jnp.float32).at[indices, cols].add(scaled)

    nblk = M // RB
    output, pmax = pl.pallas_call(
        _combine_body,
        grid=(nblk,),
        in_specs=[
            pl.BlockSpec(memory_space=pltpu.SMEM),
            pl.BlockSpec((RB, D), lambda i: (i, 0)),
            pl.BlockSpec((RB, D), lambda i: (i, 0)),
        ],
        out_specs=[
            pl.BlockSpec((RB, D), lambda i: (i, 0)),
            pl.BlockSpec((nblk,), lambda i: (0,), memory_space=pltpu.SMEM),
        ],
        out_shape=[
            jax.ShapeDtypeStruct((M, D), jnp.float32),
            jax.ShapeDtypeStruct((nblk,), jnp.float32),
        ],
    )(var_scale, var, delta)

    new_scale = (jnp.max(pmax) / 127.0).reshape(1)

    y = pl.pallas_call(
        _quant_body,
        grid=(nblk,),
        in_specs=[
            pl.BlockSpec(memory_space=pltpu.SMEM),
            pl.BlockSpec((RB, D), lambda i: (i, 0)),
        ],
        out_specs=pl.BlockSpec((RB, D), lambda i: (i, 0)),
        out_shape=jax.ShapeDtypeStruct((M, D), jnp.int8),
    )(new_scale, output)

    return (y, output, new_scale)


# speedup vs baseline: 16.4275x; 16.4275x over previous
import jax
import jax.numpy as jnp
from jax import lax
from jax.experimental import pallas as pl
from jax.experimental.pallas import tpu as pltpu
from jax.experimental.pallas import tpu_sc as plsc

M = 100000
D = 128
B = 16384
RB = 800

NC, NS = 2, 16
NW = NC * NS
CPW = D // NW
CH = 8192
NCH = B // CH

_SC_MESH = plsc.VectorSubcoreMesh(
    core_axis_name="c", subcore_axis_name="s", num_cores=NC, num_subcores=NS
)


def _sc_scatter_body(idx_hbm, upd_hbm, delta_hbm, acc, idx_v, upd_v):
    wid = lax.axis_index("s") * NC + lax.axis_index("c")
    for q in range(CPW):
        j = wid * CPW + q

        def zbody(i, _):
            for u in range(10):
                acc[pl.ds(i * 160 + u * 16, 16)] = jnp.zeros((16,), jnp.float32)
            return 0

        lax.fori_loop(0, M // 160, zbody, 0)

        for c in range(NCH):
            pltpu.sync_copy(idx_hbm.at[j, pl.ds(c * CH, CH)], idx_v)
            pltpu.sync_copy(upd_hbm.at[j, pl.ds(c * CH, CH)], upd_v)

            def sbody(k, _):
                for u in range(8):
                    off = k * 128 + u * 16
                    iv = idx_v[pl.ds(off, 16)]
                    uv = upd_v[pl.ds(off, 16)]
                    plsc.addupdate_scatter(acc, [iv], uv)
                return 0

            lax.fori_loop(0, CH // 128, sbody, 0)

        pltpu.sync_copy(acc, delta_hbm.at[j])


_sc_scatter = pl.kernel(
    _sc_scatter_body,
    out_type=jax.ShapeDtypeStruct((D, M), jnp.float32),
    mesh=_SC_MESH,
    compiler_params=pltpu.CompilerParams(needs_layout_passes=False),
    scratch_types=[
        pltpu.VMEM((M,), jnp.float32),
        pltpu.VMEM((CH,), jnp.int32),
        pltpu.VMEM((CH,), jnp.float32),
    ],
)


def _combine_body(scale_ref, var_ref, delta_ref, out_ref, pmax_ref):
    out = var_ref[...].astype(jnp.float32) * scale_ref[0] + delta_ref[...]
    out_ref[...] = out
    pmax_ref[pl.program_id(0)] = jnp.max(jnp.abs(out))


def _quant_body(scale_ref, out_ref, y_ref):
    inv = 1.0 / scale_ref[0]
    y = jnp.clip(jnp.round(out_ref[...] * inv), -128, 127)
    y_ref[...] = y.astype(jnp.int8)


def kernel(var, var_scale, indices, updates, smooth_scales):
    idx_t = indices.T
    upd_t = (updates * smooth_scales).astype(jnp.float32).T

    delta_t = _sc_scatter(idx_t, upd_t)
    delta = delta_t.T

    nblk = M // RB
    output, pmax = pl.pallas_call(
        _combine_body,
        grid=(nblk,),
        in_specs=[
            pl.BlockSpec(memory_space=pltpu.SMEM),
            pl.BlockSpec((RB, D), lambda i: (i, 0)),
            pl.BlockSpec((RB, D), lambda i: (i, 0)),
        ],
        out_specs=[
            pl.BlockSpec((RB, D), lambda i: (i, 0)),
            pl.BlockSpec((nblk,), lambda i: (0,), memory_space=pltpu.SMEM),
        ],
        out_shape=[
            jax.ShapeDtypeStruct((M, D), jnp.float32),
            jax.ShapeDtypeStruct((nblk,), jnp.float32),
        ],
    )(var_scale, var, delta)

    new_scale = (jnp.max(pmax) / 127.0).reshape(1)

    y = pl.pallas_call(
        _quant_body,
        grid=(nblk,),
        in_specs=[
            pl.BlockSpec(memory_space=pltpu.SMEM),
            pl.BlockSpec((RB, D), lambda i: (i, 0)),
        ],
        out_specs=pl.BlockSpec((RB, D), lambda i: (i, 0)),
        out_shape=jax.ShapeDtypeStruct((M, D), jnp.int8),
    )(new_scale, output)

    return (y, output, new_scale)


# speedup vs baseline: 19.6063x; 1.1935x over previous
import jax
import jax.numpy as jnp
from jax import lax
from jax.experimental import pallas as pl
from jax.experimental.pallas import tpu as pltpu
from jax.experimental.pallas import tpu_sc as plsc

M = 100000
D = 128
B = 16384
MP = 100352
RBC = 1024
RB = 800

NC, NS = 2, 16
NW = NC * NS
CPW = D // NW
CH = 8192
NCH = B // CH

_SC_MESH = plsc.VectorSubcoreMesh(
    core_axis_name="c", subcore_axis_name="s", num_cores=NC, num_subcores=NS
)


def _sc_scatter_body(idx_hbm, upd_hbm, delta_hbm, acc, idx_v, upd_v):
    wid = lax.axis_index("s") * NC + lax.axis_index("c")
    zeros16 = jnp.zeros((16,), jnp.float32)

    def zbody(i, _):
        for u in range(8):
            acc[pl.ds(i * 128 + u * 16, 16)] = zeros16
        return 0

    lax.fori_loop(0, MP // 128, zbody, 0)

    for q in range(CPW):
        j = wid * CPW + q
        pltpu.sync_copy(idx_hbm.at[j], idx_v)

        for c in range(NCH):
            pltpu.sync_copy(upd_hbm.at[j, pl.ds(c * CH, CH)], upd_v)

            def sbody(k, _):
                for u in range(8):
                    off = k * 128 + u * 16
                    iv = idx_v[pl.ds(c * CH + off, 16)]
                    uv = upd_v[pl.ds(off, 16)]
                    plsc.addupdate_scatter(acc, [iv], uv)
                return 0

            lax.fori_loop(0, CH // 128, sbody, 0)

        pltpu.sync_copy(acc, delta_hbm.at[j])

        if q < CPW - 1:
            def zsbody(k, _):
                for u in range(8):
                    iv = idx_v[pl.ds(k * 128 + u * 16, 16)]
                    plsc.store_scatter(acc, [iv], zeros16)
                return 0

            lax.fori_loop(0, B // 128, zsbody, 0)


_sc_scatter = pl.kernel(
    _sc_scatter_body,
    out_type=jax.ShapeDtypeStruct((D, MP), jnp.float32),
    mesh=_SC_MESH,
    compiler_params=pltpu.CompilerParams(needs_layout_passes=False),
    scratch_types=[
        pltpu.VMEM((MP,), jnp.float32),
        pltpu.VMEM((B,), jnp.int32),
        pltpu.VMEM((CH,), jnp.float32),
    ],
)


def _combine_body(scale_ref, var_ref, delta_ref, out_ref, pmax_ref):
    out = var_ref[...].astype(jnp.float32) * scale_ref[0] + delta_ref[...].T
    out_ref[...] = out
    row = (pl.program_id(0) * RBC
           + lax.broadcasted_iota(jnp.int32, (RBC, D), 0))
    absout = jnp.where(row < M, jnp.abs(out), 0.0)
    pmax_ref[pl.program_id(0)] = jnp.max(absout)


def _quant_body(scale_ref, out_ref, y_ref):
    inv = 1.0 / scale_ref[0]
    y = jnp.clip(jnp.round(out_ref[...] * inv), -128, 127)
    y_ref[...] = y.astype(jnp.int8)


def kernel(var, var_scale, indices, updates, smooth_scales):
    idx_t = indices.T
    upd_t = (updates * smooth_scales).astype(jnp.float32).T

    delta_t = _sc_scatter(idx_t, upd_t)

    nblk = MP // RBC
    output, pmax = pl.pallas_call(
        _combine_body,
        grid=(nblk,),
        in_specs=[
            pl.BlockSpec(memory_space=pltpu.SMEM),
            pl.BlockSpec((RBC, D), lambda i: (i, 0)),
            pl.BlockSpec((D, RBC), lambda i: (0, i)),
        ],
        out_specs=[
            pl.BlockSpec((RBC, D), lambda i: (i, 0)),
            pl.BlockSpec((nblk,), lambda i: (0,), memory_space=pltpu.SMEM),
        ],
        out_shape=[
            jax.ShapeDtypeStruct((M, D), jnp.float32),
            jax.ShapeDtypeStruct((nblk,), jnp.float32),
        ],
    )(var_scale, var, delta_t)

    new_scale = (jnp.max(pmax) / 127.0).reshape(1)

    y = pl.pallas_call(
        _quant_body,
        grid=(M // RB,),
        in_specs=[
            pl.BlockSpec(memory_space=pltpu.SMEM),
            pl.BlockSpec((RB, D), lambda i: (i, 0)),
        ],
        out_specs=pl.BlockSpec((RB, D), lambda i: (i, 0)),
        out_shape=jax.ShapeDtypeStruct((M, D), jnp.int8),
    )(new_scale, output)

    return (y, output, new_scale)


# speedup vs baseline: 28.4754x; 1.4524x over previous
import jax
import jax.numpy as jnp
from jax import lax
from jax.experimental import pallas as pl
from jax.experimental.pallas import tpu as pltpu
from jax.experimental.pallas import tpu_sc as plsc

M = 100000
D = 128
B = 16384
MP = 100352
RBC = 2048
RB = 4000

NC, NS = 2, 16
NW = NC * NS
CPW = D // NW
CH = 8192
NCH = B // CH

_SC_MESH = plsc.VectorSubcoreMesh(
    core_axis_name="c", subcore_axis_name="s", num_cores=NC, num_subcores=NS
)


def _sc_scatter_body(idx_hbm, upd_hbm, delta_hbm, acc, idx_v, upd_v):
    wid = lax.axis_index("s") * NC + lax.axis_index("c")
    zeros16 = jnp.zeros((16,), jnp.float32)

    def zbody(i, _):
        for u in range(8):
            acc[pl.ds(i * 128 + u * 16, 16)] = zeros16
        return 0

    lax.fori_loop(0, MP // 128, zbody, 0)

    for q in range(CPW):
        j = wid * CPW + q
        pltpu.sync_copy(idx_hbm.at[j], idx_v)

        for c in range(NCH):
            pltpu.sync_copy(upd_hbm.at[j, pl.ds(c * CH, CH)], upd_v)

            def sbody(k, _):
                for u in range(8):
                    off = k * 128 + u * 16
                    iv = idx_v[pl.ds(c * CH + off, 16)]
                    uv = upd_v[pl.ds(off, 16)]
                    plsc.addupdate_scatter(acc, [iv], uv)
                return 0

            lax.fori_loop(0, CH // 128, sbody, 0)

        pltpu.sync_copy(acc, delta_hbm.at[j])


_sc_scatter = pl.kernel(
    _sc_scatter_body,
    out_type=jax.ShapeDtypeStruct((D, MP), jnp.float32),
    mesh=_SC_MESH,
    compiler_params=pltpu.CompilerParams(needs_layout_passes=False),
    scratch_types=[
        pltpu.VMEM((MP,), jnp.float32),
        pltpu.VMEM((B,), jnp.int32),
        pltpu.VMEM((CH,), jnp.float32),
    ],
)


def _combine_body(scale_ref, a_ref, var_ref, p_ref, out_ref, pmax_ref):
    dt = lax.dot_general(
        p_ref[...], a_ref[...], (((0,), (1,)), ((), ())),
        preferred_element_type=jnp.float32,
    )
    out = var_ref[...].astype(jnp.float32) * scale_ref[0] + dt
    out_ref[...] = out
    row = (pl.program_id(0) * RBC
           + lax.broadcasted_iota(jnp.int32, (RBC, D), 0))
    absout = jnp.where(row < M, jnp.abs(out), 0.0)
    pmax_ref[pl.program_id(0)] = jnp.max(absout)


def _quant_body(scale_ref, out_ref, y_ref):
    inv = 1.0 / scale_ref[0]
    y = jnp.clip(jnp.round(out_ref[...] * inv), -128, 127)
    y_ref[...] = y.astype(jnp.int8)


def kernel(var, var_scale, indices, updates, smooth_scales):
    idx_t = indices.T
    upd_t = (updates * smooth_scales).astype(jnp.float32).T

    delta_t = _sc_scatter(idx_t, upd_t)

    sub = -(jnp.arange(1, D) % 4 != 0).astype(jnp.float32)
    a_mat = jnp.eye(D, dtype=jnp.float32) + jnp.diag(sub, -1)

    nblk = MP // RBC
    output, pmax = pl.pallas_call(
        _combine_body,
        grid=(nblk,),
        in_specs=[
            pl.BlockSpec(memory_space=pltpu.SMEM),
            pl.BlockSpec((D, D), lambda i: (0, 0)),
            pl.BlockSpec((RBC, D), lambda i: (i, 0)),
            pl.BlockSpec((D, RBC), lambda i: (0, i)),
        ],
        out_specs=[
            pl.BlockSpec((RBC, D), lambda i: (i, 0)),
            pl.BlockSpec((nblk,), lambda i: (0,), memory_space=pltpu.SMEM),
        ],
        out_shape=[
            jax.ShapeDtypeStruct((M, D), jnp.float32),
            jax.ShapeDtypeStruct((nblk,), jnp.float32),
        ],
    )(var_scale, a_mat, var, delta_t)

    new_scale = (jnp.max(pmax) / 127.0).reshape(1)

    y = pl.pallas_call(
        _quant_body,
        grid=(M // RB,),
        in_specs=[
            pl.BlockSpec(memory_space=pltpu.SMEM),
            pl.BlockSpec((RB, D), lambda i: (i, 0)),
        ],
        out_specs=pl.BlockSpec((RB, D), lambda i: (i, 0)),
        out_shape=jax.ShapeDtypeStruct((M, D), jnp.int8),
    )(new_scale, output)

    return (y, output, new_scale)
